# baseline (speedup 1.0000x reference)
import functools

import jax
import jax.numpy as jnp
import numpy as np
from jax import lax
from jax.experimental import pallas as pl
from jax.experimental.pallas import tpu as pltpu
from jax.experimental.pallas import tpu_sc as plsc

_N = 4096
_D = 32
_K = 8192
_NB = 1024
_KB = 1024
_C = 128
_NCH = _KB // _C
_COMMIT = 0.25
_BIG = 3.0e38


def _argmin_body(flat_ref, emb_ref, x2_ref, e2_ref,
                 qidx_ref, loss_ref, val_ref, idc_ref, acc_ref):
    n = pl.program_id(0)
    k = pl.program_id(1)
    s2 = lax.dot_general(
        2.0 * flat_ref[...], emb_ref[...],
        dimension_numbers=(((1,), (1,)), ((), ())),
        preferred_element_type=jnp.float32,
    )
    x2 = x2_ref[...]
    e2 = e2_ref[0]

    @pl.when(k == 0)
    def _():
        val_ref[...] = jnp.full((_NB, _C), _BIG, jnp.float32)
        idc_ref[...] = jnp.zeros((_NB, _C), jnp.float32)

    for c in range(_NCH):
        sl = slice(c * _C, (c + 1) * _C)
        d = (x2 + e2[:, sl]) - s2[:, sl]
        v = val_ref[...]
        m = d < v
        val_ref[...] = jnp.where(m, d, v)
        idc_ref[...] = jnp.where(m, jnp.float32(1.0) * (k * _NCH + c),
                                 idc_ref[...])

    @pl.when(k == pl.num_programs(1) - 1)
    def _():
        v = val_ref[...]
        rmin = jnp.min(v, axis=1, keepdims=True)
        lane = lax.broadcasted_iota(jnp.int32, (_NB, _C), 1).astype(jnp.float32)
        gidx = idc_ref[...] * _C + lane
        cand = jnp.where(v == rmin, gidx, _BIG)
        gmin = jnp.min(cand, axis=1, keepdims=True)
        qidx_ref[...] = gmin.astype(jnp.int32)
        bsum = jnp.sum(rmin)

        @pl.when(n == 0)
        def _():
            acc_ref[0] = bsum

        @pl.when(n > 0)
        def _():
            acc_ref[0] = acc_ref[0] + bsum

        @pl.when(n == pl.num_programs(0) - 1)
        def _():
            loss_ref[0, 0] = _COMMIT * (acc_ref[0] * (1.0 / (_N * _D)))


def _argmin_call(flat, embed, x2, e2r):
    return pl.pallas_call(
        _argmin_body,
        grid=(_N // _NB, _K // _KB),
        in_specs=[
            pl.BlockSpec((_NB, _D), lambda n, k: (n, 0)),
            pl.BlockSpec((_KB, _D), lambda n, k: (k, 0)),
            pl.BlockSpec((_NB, 1), lambda n, k: (n, 0)),
            pl.BlockSpec((1, 1, _KB), lambda n, k: (k, 0, 0)),
        ],
        out_specs=[
            pl.BlockSpec((_NB, 1), lambda n, k: (n, 0)),
            pl.BlockSpec(memory_space=pltpu.SMEM),
        ],
        out_shape=[
            jax.ShapeDtypeStruct((_N, 1), jnp.int32),
            jax.ShapeDtypeStruct((1, 1), jnp.float32),
        ],
        scratch_shapes=[
            pltpu.VMEM((_NB, _C), jnp.float32),
            pltpu.VMEM((_NB, _C), jnp.float32),
            pltpu.SMEM((1,), jnp.float32),
        ],
    )(flat, embed, x2, e2r)


_NC = 2
_NS = 16
_NW = _NC * _NS
_BPW = _N // _NW
_CPW = _K // _NS


def _sc_body(idx_hbm, emb_hbm, zeros_hbm, ones_hbm,
             zq_hbm, counts_hbm,
             idx_v, rows_v, ones_v, counts_sh, sem):
    c = lax.axis_index("c")
    s = lax.axis_index("s")
    w = s * _NC + c
    base = w * _BPW
    pltpu.sync_copy(idx_hbm.at[pl.ds(base, _BPW)], idx_v)
    pltpu.async_copy(emb_hbm.at[idx_v], rows_v, sem).wait()
    pltpu.sync_copy(rows_v, zq_hbm.at[pl.ds(base, _BPW)])
    pltpu.sync_copy(ones_hbm, ones_v)

    @pl.when(s == 0)
    def _():
        pltpu.sync_copy(zeros_hbm, counts_sh)

    plsc.subcore_barrier()
    pltpu.sync_copy(ones_v, counts_sh.at[idx_v], add=True)
    plsc.subcore_barrier()
    pltpu.sync_copy(counts_sh.at[pl.ds(s * _CPW, _CPW)],
                    counts_hbm.at[c, pl.ds(s * _CPW, _CPW)])


@functools.cache
def _sc_gather_hist():
    return pl.kernel(
        _sc_body,
        mesh=plsc.VectorSubcoreMesh(core_axis_name="c", subcore_axis_name="s"),
        compiler_params=pltpu.CompilerParams(use_tc_tiling_on_sc=False),
        out_type=[
            jax.ShapeDtypeStruct((_N, _D), jnp.float32),
            jax.ShapeDtypeStruct((_NC, _K), jnp.float32),
        ],
        scratch_types=[
            pltpu.VMEM((_BPW,), jnp.int32),
            pltpu.VMEM((_BPW, _D), jnp.float32),
            pltpu.VMEM((_BPW,), jnp.float32),
            pltpu.VMEM_SHARED((_K,), jnp.float32),
            pltpu.SemaphoreType.DMA,
        ],
    )


def _perp_body(cnt_ref, perp_ref):
    cnt = cnt_ref[0] + cnt_ref[1]
    p = cnt * (1.0 / _N)
    ent = -jnp.sum(p * jnp.log(p + 1e-10))
    perp_ref[0, 0] = jnp.exp(ent)


def _perp_call(counts3):
    return pl.pallas_call(
        _perp_body,
        in_specs=[pl.BlockSpec(memory_space=pltpu.VMEM)],
        out_specs=pl.BlockSpec(memory_space=pltpu.SMEM),
        out_shape=jax.ShapeDtypeStruct((1, 1), jnp.float32),
    )(counts3)


_KLDIV_VAL = np.log(float(_K)) * (_N / 4)


def kernel(inputs, embed):
    x = jnp.swapaxes(inputs, 1, -1)
    input_shape = x.shape
    flat = x.reshape(_N, _D)
    x2 = jnp.sum(flat * flat, axis=1, keepdims=True)
    e2 = jnp.sum(embed * embed, axis=1)
    e2r = e2.reshape(_K // _KB, 1, _KB)

    qidx, loss = _argmin_call(flat, embed, x2, e2r)

    zeros = jnp.zeros((_K,), jnp.float32)
    ones = jnp.ones((_BPW,), jnp.float32)
    zq, counts = _sc_gather_hist()(qidx.reshape(_N), embed, zeros, ones)

    perp = _perp_call(counts.reshape(_NC, _K // 128, 128))

    zq_r = zq.astype(jnp.bfloat16).astype(jnp.float32)
    z_q_st = flat + (zq_r - flat)
    z_q_out = jnp.swapaxes(z_q_st.reshape(input_shape), 1, -1)
    kldiv_r = jnp.full((inputs.shape[0], 1), _KLDIV_VAL, jnp.float32)
    return (z_q_out, loss[0, 0], kldiv_r, perp[0, 0])


# speedup vs baseline: 1.1681x; 1.1681x over previous
import functools

import jax
import jax.numpy as jnp
import numpy as np
from jax import lax
from jax.experimental import pallas as pl
from jax.experimental.pallas import tpu as pltpu
from jax.experimental.pallas import tpu_sc as plsc

_N = 4096
_D = 32
_K = 8192
_NB = 1024
_KB = 1024
_C = 128
_NCH = _KB // _C
_COMMIT = 0.25
_BIG = 3.0e38


def _argmin_body(flat_ref, emb_ref, x2_ref, e2_ref,
                 qidx_ref, loss_ref, val_ref, idc_ref, acc_ref):
    n = pl.program_id(0)
    k = pl.program_id(1)
    flat2 = 2.0 * flat_ref[...]

    def _tile_dot(h):
        return lax.dot_general(
            flat2, emb_ref[h * 256:(h + 1) * 256, :],
            dimension_numbers=(((1,), (1,)), ((), ())),
            preferred_element_type=jnp.float32,
        )

    x2 = x2_ref[...]
    e2 = e2_ref[0]

    @pl.when(k == 0)
    def _():
        val_ref[...] = jnp.full((_NB, _C), _BIG, jnp.float32)
        idc_ref[...] = jnp.zeros((_NB, _C), jnp.float32)

    s2p = {0: _tile_dot(0)}
    for c in range(_NCH):
        if c % 2 == 0 and c // 2 + 1 in range(_KB // 256):
            s2p[c // 2 + 1] = _tile_dot(c // 2 + 1)
        sl = slice(c * _C, (c + 1) * _C)
        psl = slice((c % 2) * _C, (c % 2) * _C + _C)
        d = (x2 + e2[:, sl]) - s2p[c // 2][:, psl]
        v = val_ref[...]
        m = d < v
        val_ref[...] = jnp.where(m, d, v)
        idc_ref[...] = jnp.where(m, jnp.float32(1.0) * (k * _NCH + c),
                                 idc_ref[...])

    @pl.when(k == pl.num_programs(1) - 1)
    def _():
        v = val_ref[...]
        rmin = jnp.min(v, axis=1, keepdims=True)
        lane = lax.broadcasted_iota(jnp.int32, (_NB, _C), 1).astype(jnp.float32)
        gidx = idc_ref[...] * _C + lane
        cand = jnp.where(v == rmin, gidx, _BIG)
        gmin = jnp.min(cand, axis=1, keepdims=True)
        qidx_ref[...] = gmin.astype(jnp.int32)
        bsum = jnp.sum(rmin)

        @pl.when(n == 0)
        def _():
            acc_ref[0] = bsum

        @pl.when(n > 0)
        def _():
            acc_ref[0] = acc_ref[0] + bsum

        @pl.when(n == pl.num_programs(0) - 1)
        def _():
            loss_ref[0, 0] = _COMMIT * (acc_ref[0] * (1.0 / (_N * _D)))


def _argmin_call(flat, embed, x2, e2r):
    return pl.pallas_call(
        _argmin_body,
        grid=(_N // _NB, _K // _KB),
        in_specs=[
            pl.BlockSpec((_NB, _D), lambda n, k: (n, 0)),
            pl.BlockSpec((_KB, _D), lambda n, k: (k, 0)),
            pl.BlockSpec((_NB, 1), lambda n, k: (n, 0)),
            pl.BlockSpec((1, 1, _KB), lambda n, k: (k, 0, 0)),
        ],
        out_specs=[
            pl.BlockSpec((_NB, 1), lambda n, k: (n, 0)),
            pl.BlockSpec(memory_space=pltpu.SMEM),
        ],
        out_shape=[
            jax.ShapeDtypeStruct((_N, 1), jnp.int32),
            jax.ShapeDtypeStruct((1, 1), jnp.float32),
        ],
        scratch_shapes=[
            pltpu.VMEM((_NB, _C), jnp.float32),
            pltpu.VMEM((_NB, _C), jnp.float32),
            pltpu.SMEM((1,), jnp.float32),
        ],
    )(flat, embed, x2, e2r)


_NC = 2
_NS = 16
_NW = _NC * _NS
_BPW = _N // _NW
_CPW = _K // _NS


def _sc_body(idx_hbm, emb_hbm, zeros_hbm, ones_hbm,
             zq_hbm, counts_hbm,
             idx_v, rows_v, ones_v, counts_sh, sem):
    c = lax.axis_index("c")
    s = lax.axis_index("s")
    w = s * _NC + c
    base = w * _BPW
    pltpu.sync_copy(idx_hbm.at[pl.ds(base, _BPW)], idx_v)
    pltpu.async_copy(emb_hbm.at[idx_v], rows_v, sem).wait()
    pltpu.sync_copy(rows_v, zq_hbm.at[pl.ds(base, _BPW)])
    pltpu.sync_copy(ones_hbm, ones_v)

    @pl.when(s == 0)
    def _():
        pltpu.sync_copy(zeros_hbm, counts_sh)

    plsc.subcore_barrier()
    pltpu.sync_copy(ones_v, counts_sh.at[idx_v], add=True)
    plsc.subcore_barrier()
    pltpu.sync_copy(counts_sh.at[pl.ds(s * _CPW, _CPW)],
                    counts_hbm.at[c, pl.ds(s * _CPW, _CPW)])


@functools.cache
def _sc_gather_hist():
    return pl.kernel(
        _sc_body,
        mesh=plsc.VectorSubcoreMesh(core_axis_name="c", subcore_axis_name="s"),
        compiler_params=pltpu.CompilerParams(use_tc_tiling_on_sc=False),
        out_type=[
            jax.ShapeDtypeStruct((_N, _D), jnp.float32),
            jax.ShapeDtypeStruct((_NC, _K), jnp.float32),
        ],
        scratch_types=[
            pltpu.VMEM((_BPW,), jnp.int32),
            pltpu.VMEM((_BPW, _D), jnp.float32),
            pltpu.VMEM((_BPW,), jnp.float32),
            pltpu.VMEM_SHARED((_K,), jnp.float32),
            pltpu.SemaphoreType.DMA,
        ],
    )


def _perp_body(cnt_ref, perp_ref):
    cnt = cnt_ref[0] + cnt_ref[1]
    p = cnt * (1.0 / _N)
    ent = -jnp.sum(p * jnp.log(p + 1e-10))
    perp_ref[0, 0] = jnp.exp(ent)


def _perp_call(counts3):
    return pl.pallas_call(
        _perp_body,
        in_specs=[pl.BlockSpec(memory_space=pltpu.VMEM)],
        out_specs=pl.BlockSpec(memory_space=pltpu.SMEM),
        out_shape=jax.ShapeDtypeStruct((1, 1), jnp.float32),
    )(counts3)


_KLDIV_VAL = np.log(float(_K)) * (_N / 4)


def kernel(inputs, embed):
    x = jnp.swapaxes(inputs, 1, -1)
    input_shape = x.shape
    flat = x.reshape(_N, _D)
    x2 = jnp.sum(flat * flat, axis=1, keepdims=True)
    e2 = jnp.sum(embed * embed, axis=1)
    e2r = e2.reshape(_K // _KB, 1, _KB)

    qidx, loss = _argmin_call(flat, embed, x2, e2r)

    zeros = jnp.zeros((_K,), jnp.float32)
    ones = jnp.ones((_BPW,), jnp.float32)
    zq, counts = _sc_gather_hist()(qidx.reshape(_N), embed, zeros, ones)

    perp = _perp_call(counts.reshape(_NC, _K // 128, 128))

    zq_r = zq.astype(jnp.bfloat16).astype(jnp.float32)
    z_q_st = flat + (zq_r - flat)
    z_q_out = jnp.swapaxes(z_q_st.reshape(input_shape), 1, -1)
    kldiv_r = jnp.full((inputs.shape[0], 1), _KLDIV_VAL, jnp.float32)
    return (z_q_out, loss[0, 0], kldiv_r, perp[0, 0])


# speedup vs baseline: 1.1933x; 1.0216x over previous
import functools

import jax
import jax.numpy as jnp
import numpy as np
from jax import lax
from jax.experimental import pallas as pl
from jax.experimental.pallas import tpu as pltpu
from jax.experimental.pallas import tpu_sc as plsc

_N = 4096
_D = 32
_K = 8192
_NB = 1024
_KB = 1024
_C = 128
_NCH = _KB // _C
_COMMIT = 0.25
_BIG = 3.0e38


def _argmin_body(flat_ref, emb_ref, x2_ref, e2_ref,
                 qidx_ref, loss_ref, val_ref, idc_ref, acc_ref):
    n = pl.program_id(0)
    k = pl.program_id(1)
    flat2 = 2.0 * flat_ref[...]

    def _tile_dot(h):
        return lax.dot_general(
            flat2, emb_ref[h * 256:(h + 1) * 256, :],
            dimension_numbers=(((1,), (1,)), ((), ())),
            preferred_element_type=jnp.float32,
        )

    x2 = x2_ref[...]
    e2 = e2_ref[0]

    @pl.when(k == 0)
    def _():
        val_ref[...] = jnp.full((_NB, _C), _BIG, jnp.float32)
        idc_ref[...] = jnp.zeros((_NB, _C), jnp.float32)

    s2p = {0: _tile_dot(0)}
    for c in range(_NCH):
        if c % 2 == 0 and c // 2 + 1 in range(_KB // 256):
            s2p[c // 2 + 1] = _tile_dot(c // 2 + 1)
        sl = slice(c * _C, (c + 1) * _C)
        psl = slice((c % 2) * _C, (c % 2) * _C + _C)
        d = (x2 + e2[:, sl]) - s2p[c // 2][:, psl]
        v = val_ref[...]
        m = d < v
        val_ref[...] = jnp.where(m, d, v)
        idc_ref[...] = jnp.where(m, jnp.float32(1.0) * (k * _NCH + c),
                                 idc_ref[...])

    @pl.when(k == pl.num_programs(1) - 1)
    def _():
        v = val_ref[...]
        rmin = jnp.min(v, axis=1, keepdims=True)
        lane = lax.broadcasted_iota(jnp.int32, (_NB, _C), 1).astype(jnp.float32)
        gidx = idc_ref[...] * _C + lane
        cand = jnp.where(v == rmin, gidx, _BIG)
        gmin = jnp.min(cand, axis=1, keepdims=True)
        qidx_ref[...] = gmin.astype(jnp.int32)
        bsum = jnp.sum(rmin)

        @pl.when(n == 0)
        def _():
            acc_ref[0] = bsum

        @pl.when(n > 0)
        def _():
            acc_ref[0] = acc_ref[0] + bsum

        @pl.when(n == pl.num_programs(0) - 1)
        def _():
            loss_ref[0, 0] = _COMMIT * (acc_ref[0] * (1.0 / (_N * _D)))


def _argmin_call(flat, embed, x2, e2r):
    return pl.pallas_call(
        _argmin_body,
        grid=(_N // _NB, _K // _KB),
        in_specs=[
            pl.BlockSpec((_NB, _D), lambda n, k: (n, 0)),
            pl.BlockSpec((_KB, _D), lambda n, k: (k, 0)),
            pl.BlockSpec((_NB, 1), lambda n, k: (n, 0)),
            pl.BlockSpec((1, 1, _KB), lambda n, k: (k, 0, 0)),
        ],
        out_specs=[
            pl.BlockSpec((_NB, 1), lambda n, k: (n, 0)),
            pl.BlockSpec(memory_space=pltpu.SMEM),
        ],
        out_shape=[
            jax.ShapeDtypeStruct((_N, 1), jnp.int32),
            jax.ShapeDtypeStruct((1, 1), jnp.float32),
        ],
        scratch_shapes=[
            pltpu.VMEM((_NB, _C), jnp.float32),
            pltpu.VMEM((_NB, _C), jnp.float32),
            pltpu.SMEM((1,), jnp.float32),
        ],
    )(flat, embed, x2, e2r)


_NS = 16
_BPW = _N // _NS
_CPW = _K // _NS
_TPAD = 4104


def _sc_body(idx_hbm, emb_hbm, zeros_hbm, ones_hbm, table_hbm,
             zq_hbm, perp_hbm,
             idx_v, rows_v, ones_v, cnt_v, table_v, ent_v, acc_v,
             counts_sh, ent_sh, sem):
    s = lax.axis_index("s")
    base = s * _BPW
    pltpu.sync_copy(idx_hbm.at[pl.ds(base, _BPW)], idx_v)
    pltpu.async_copy(emb_hbm.at[idx_v], rows_v, sem).wait()
    pltpu.sync_copy(rows_v, zq_hbm.at[pl.ds(base, _BPW)])
    pltpu.sync_copy(ones_hbm, ones_v)
    pltpu.sync_copy(table_hbm, table_v)

    @pl.when(s == 0)
    def _():
        pltpu.sync_copy(zeros_hbm, counts_sh)

    plsc.subcore_barrier()
    pltpu.sync_copy(ones_v, counts_sh.at[idx_v], add=True)
    plsc.subcore_barrier()
    pltpu.sync_copy(counts_sh.at[pl.ds(s * _CPW, _CPW)], cnt_v)
    acc = jnp.zeros((16,), jnp.float32)
    for j in range(_CPW // 16):
        ci = cnt_v[pl.ds(j * 16, 16)].astype(jnp.int32)
        acc = acc + plsc.load_gather(table_v, [ci])
    ent_v[...] = acc
    pltpu.sync_copy(ent_v, ent_sh.at[s])
    plsc.subcore_barrier()

    @pl.when(s == 0)
    def _():
        pltpu.sync_copy(ent_sh, acc_v)
        tot = jnp.zeros((16,), jnp.float32)
        for r in range(_NS):
            tot = tot + acc_v[r]
        ent = jnp.sum(tot)
        ent_v[...] = jnp.exp(jnp.broadcast_to(ent, (16,)))
        pltpu.sync_copy(ent_v, perp_hbm)


@functools.cache
def _sc_gather_hist():
    return pl.kernel(
        _sc_body,
        mesh=plsc.VectorSubcoreMesh(
            core_axis_name="c", subcore_axis_name="s", num_cores=1),
        compiler_params=pltpu.CompilerParams(
            use_tc_tiling_on_sc=False, needs_layout_passes=False),
        out_type=[
            jax.ShapeDtypeStruct((_N, _D), jnp.float32),
            jax.ShapeDtypeStruct((16,), jnp.float32),
        ],
        scratch_types=[
            pltpu.VMEM((_BPW,), jnp.int32),
            pltpu.VMEM((_BPW, _D), jnp.float32),
            pltpu.VMEM((_BPW,), jnp.float32),
            pltpu.VMEM((_CPW,), jnp.float32),
            pltpu.VMEM((_TPAD,), jnp.float32),
            pltpu.VMEM((16,), jnp.float32),
            pltpu.VMEM((_NS, 16), jnp.float32),
            pltpu.VMEM_SHARED((_K,), jnp.float32),
            pltpu.VMEM_SHARED((_NS, 16), jnp.float32),
            pltpu.SemaphoreType.DMA,
        ],
    )


_KLDIV_VAL = np.log(float(_K)) * (_N / 4)


def kernel(inputs, embed):
    x = jnp.swapaxes(inputs, 1, -1)
    input_shape = x.shape
    flat = x.reshape(_N, _D)
    x2 = jnp.sum(flat * flat, axis=1, keepdims=True)
    e2 = jnp.sum(embed * embed, axis=1)
    e2r = e2.reshape(_K // _KB, 1, _KB)

    qidx, loss = _argmin_call(flat, embed, x2, e2r)

    zeros = jnp.zeros((_K,), jnp.float32)
    ones = jnp.ones((_BPW,), jnp.float32)
    cgrid = jnp.arange(_TPAD, dtype=jnp.float32) * (1.0 / _N)
    table = -(cgrid * jnp.log(cgrid + 1e-10)).at[0].set(0.0)
    zq, perp_v = _sc_gather_hist()(qidx.reshape(_N), embed, zeros, ones, table)
    perp = perp_v[0].reshape(1, 1)

    zq_r = zq.astype(jnp.bfloat16).astype(jnp.float32)
    z_q_st = flat + (zq_r - flat)
    z_q_out = jnp.swapaxes(z_q_st.reshape(input_shape), 1, -1)
    kldiv_r = jnp.full((inputs.shape[0], 1), _KLDIV_VAL, jnp.float32)
    return (z_q_out, loss[0, 0], kldiv_r, perp[0, 0])


# speedup vs baseline: 1.3025x; 1.0915x over previous
import functools

import jax
import jax.numpy as jnp
import numpy as np
from jax import lax
from jax.experimental import pallas as pl
from jax.experimental.pallas import tpu as pltpu
from jax.experimental.pallas import tpu_sc as plsc

_N = 4096
_D = 32
_K = 8192
_NB = 4096
_KB = 1024
_C = 128
_NCH = _KB // _C
_COMMIT = 0.25
_BIG = 3.0e38


def _argmin_body(flat_ref, emb_ref, x2_ref, e2_ref,
                 qidx_ref, loss_ref, val_ref, idc_ref, acc_ref):
    n = pl.program_id(0)
    k = pl.program_id(1)
    flat2 = 2.0 * flat_ref[...]

    def _tile_dot(h):
        return lax.dot_general(
            flat2, emb_ref[h * 256:(h + 1) * 256, :],
            dimension_numbers=(((1,), (1,)), ((), ())),
            preferred_element_type=jnp.float32,
        )

    x2 = x2_ref[...]
    e2 = e2_ref[0]

    @pl.when(k == 0)
    def _():
        val_ref[...] = jnp.full((_NB, _C), _BIG, jnp.float32)
        idc_ref[...] = jnp.zeros((_NB, _C), jnp.float32)

    s2p = {0: _tile_dot(0)}
    for c in range(_NCH):
        if c % 2 == 0 and c // 2 + 1 in range(_KB // 256):
            s2p[c // 2 + 1] = _tile_dot(c // 2 + 1)
        sl = slice(c * _C, (c + 1) * _C)
        psl = slice((c % 2) * _C, (c % 2) * _C + _C)
        d = (x2 + e2[:, sl]) - s2p[c // 2][:, psl]
        v = val_ref[...]
        m = d < v
        val_ref[...] = jnp.where(m, d, v)
        idc_ref[...] = jnp.where(m, jnp.float32(1.0) * (k * _NCH + c),
                                 idc_ref[...])

    @pl.when(k == pl.num_programs(1) - 1)
    def _():
        v = val_ref[...]
        rmin = jnp.min(v, axis=1, keepdims=True)
        lane = lax.broadcasted_iota(jnp.int32, (_NB, _C), 1).astype(jnp.float32)
        gidx = idc_ref[...] * _C + lane
        cand = jnp.where(v == rmin, gidx, _BIG)
        gmin = jnp.min(cand, axis=1, keepdims=True)
        qidx_ref[...] = gmin.astype(jnp.int32)
        bsum = jnp.sum(rmin)

        @pl.when(n == 0)
        def _():
            acc_ref[0] = bsum

        @pl.when(n > 0)
        def _():
            acc_ref[0] = acc_ref[0] + bsum

        @pl.when(n == pl.num_programs(0) - 1)
        def _():
            loss_ref[0, 0] = _COMMIT * (acc_ref[0] * (1.0 / (_N * _D)))


def _argmin_call(flat, embed, x2, e2r):
    return pl.pallas_call(
        _argmin_body,
        grid=(_N // _NB, _K // _KB),
        in_specs=[
            pl.BlockSpec((_NB, _D), lambda n, k: (n, 0)),
            pl.BlockSpec((_KB, _D), lambda n, k: (k, 0)),
            pl.BlockSpec((_NB, 1), lambda n, k: (n, 0)),
            pl.BlockSpec((1, 1, _KB), lambda n, k: (k, 0, 0)),
        ],
        out_specs=[
            pl.BlockSpec((_NB, 1), lambda n, k: (n, 0)),
            pl.BlockSpec(memory_space=pltpu.SMEM),
        ],
        out_shape=[
            jax.ShapeDtypeStruct((_N, 1), jnp.int32),
            jax.ShapeDtypeStruct((1, 1), jnp.float32),
        ],
        scratch_shapes=[
            pltpu.VMEM((_NB, _C), jnp.float32),
            pltpu.VMEM((_NB, _C), jnp.float32),
            pltpu.SMEM((1,), jnp.float32),
        ],
    )(flat, embed, x2, e2r)


_NS = 16
_BPW = _N // _NS
_CPW = _K // _NS
_TPAD = 4104


def _sc_body(idx_hbm, emb_hbm, zeros_hbm, ones_hbm, table_hbm,
             zq_hbm, perp_hbm,
             idx_v, rows_v, ones_v, cnt_v, table_v, ent_v, acc_v,
             counts_sh, ent_sh, sem, sem_w, sem_a, sem_b):
    s = lax.axis_index("s")
    base = s * _BPW

    @pl.when(s == 0)
    def _():
        pltpu.sync_copy(zeros_hbm, counts_sh)

    pltpu.sync_copy(idx_hbm.at[pl.ds(base, _BPW)], idx_v)
    gat = pltpu.async_copy(emb_hbm.at[idx_v], rows_v, sem)
    h_ones = pltpu.async_copy(ones_hbm, ones_v, sem_a)
    h_tab = pltpu.async_copy(table_hbm, table_v, sem_b)
    gat.wait()
    zqw = pltpu.async_copy(rows_v, zq_hbm.at[pl.ds(base, _BPW)], sem_w)
    h_ones.wait()
    h_tab.wait()
    plsc.subcore_barrier()
    pltpu.sync_copy(ones_v, counts_sh.at[idx_v], add=True)
    plsc.subcore_barrier()
    pltpu.sync_copy(counts_sh.at[pl.ds(s * _CPW, _CPW)], cnt_v)
    acc = jnp.zeros((16,), jnp.float32)
    for j in range(_CPW // 16):
        ci = cnt_v[pl.ds(j * 16, 16)].astype(jnp.int32)
        acc = acc + plsc.load_gather(table_v, [ci])
    ent_v[...] = acc
    pltpu.sync_copy(ent_v, ent_sh.at[s])
    plsc.subcore_barrier()

    @pl.when(s == 0)
    def _():
        pltpu.sync_copy(ent_sh, acc_v)
        tot = jnp.zeros((16,), jnp.float32)
        for r in range(_NS):
            tot = tot + acc_v[r]
        ent = jnp.sum(tot)
        ent_v[...] = jnp.exp(jnp.broadcast_to(ent, (16,)))
        pltpu.sync_copy(ent_v, perp_hbm)

    zqw.wait()


@functools.cache
def _sc_gather_hist():
    return pl.kernel(
        _sc_body,
        mesh=plsc.VectorSubcoreMesh(
            core_axis_name="c", subcore_axis_name="s", num_cores=1),
        compiler_params=pltpu.CompilerParams(
            use_tc_tiling_on_sc=False, needs_layout_passes=False),
        out_type=[
            jax.ShapeDtypeStruct((_N, _D), jnp.float32),
            jax.ShapeDtypeStruct((16,), jnp.float32),
        ],
        scratch_types=[
            pltpu.VMEM((_BPW,), jnp.int32),
            pltpu.VMEM((_BPW, _D), jnp.float32),
            pltpu.VMEM((_BPW,), jnp.float32),
            pltpu.VMEM((_CPW,), jnp.float32),
            pltpu.VMEM((_TPAD,), jnp.float32),
            pltpu.VMEM((16,), jnp.float32),
            pltpu.VMEM((_NS, 16), jnp.float32),
            pltpu.VMEM_SHARED((_K,), jnp.float32),
            pltpu.VMEM_SHARED((_NS, 16), jnp.float32),
            pltpu.SemaphoreType.DMA,
            pltpu.SemaphoreType.DMA,
            pltpu.SemaphoreType.DMA,
            pltpu.SemaphoreType.DMA,
        ],
    )


_KLDIV_VAL = np.log(float(_K)) * (_N / 4)


def kernel(inputs, embed):
    x = jnp.swapaxes(inputs, 1, -1)
    input_shape = x.shape
    flat = x.reshape(_N, _D)
    x2 = jnp.sum(flat * flat, axis=1, keepdims=True)
    e2 = jnp.sum(embed * embed, axis=1)
    e2r = e2.reshape(_K // _KB, 1, _KB)

    qidx, loss = _argmin_call(flat, embed, x2, e2r)

    zeros = jnp.zeros((_K,), jnp.float32)
    ones = jnp.ones((_BPW,), jnp.float32)
    cgrid = jnp.arange(_TPAD, dtype=jnp.float32) * (1.0 / _N)
    table = -(cgrid * jnp.log(cgrid + 1e-10)).at[0].set(0.0)
    zq, perp_v = _sc_gather_hist()(qidx.reshape(_N), embed, zeros, ones, table)
    perp = perp_v[0].reshape(1, 1)

    zq_r = zq.astype(jnp.bfloat16).astype(jnp.float32)
    z_q_out = jnp.swapaxes(zq_r.reshape(input_shape), 1, -1)
    kldiv_r = jnp.full((inputs.shape[0], 1), _KLDIV_VAL, jnp.float32)
    return (z_q_out, loss[0, 0], kldiv_r, perp[0, 0])


# speedup vs baseline: 1.3400x; 1.0288x over previous
import functools

import jax
import jax.numpy as jnp
import numpy as np
from jax import lax
from jax.experimental import pallas as pl
from jax.experimental.pallas import tpu as pltpu
from jax.experimental.pallas import tpu_sc as plsc

_N = 4096
_D = 32
_K = 8192
_NB = 4096
_KB = 1024
_C = 128
_NCH = _KB // _C
_COMMIT = 0.25
_BIG = 3.0e38


def _argmin_body(flat_ref, emb_ref, x2_ref, e2_ref, qidx_ref, loss_ref):
    flat2 = 2.0 * flat_ref[...]

    def _tile_dot(h):
        return lax.dot_general(
            flat2, emb_ref[h * 256:(h + 1) * 256, :],
            dimension_numbers=(((1,), (1,)), ((), ())),
            preferred_element_type=jnp.float32,
        )

    x2 = x2_ref[...]
    ntiles = _K // 256
    val = None
    s2p = {0: _tile_dot(0)}
    for c in range(_K // _C):
        if c % 2 == 0 and c // 2 + 1 < ntiles:
            s2p[c // 2 + 1] = _tile_dot(c // 2 + 1)
        e2c = e2_ref[c // _NCH, :, (c % _NCH) * _C:(c % _NCH) * _C + _C]
        psl = slice((c % 2) * _C, (c % 2) * _C + _C)
        d = (x2 + e2c) - s2p[c // 2][:, psl]
        if val is None:
            val, idc = d, jnp.zeros((_NB, _C), jnp.float32)
        else:
            m = d < val
            val = jnp.where(m, d, val)
            idc = jnp.where(m, jnp.float32(1.0) * c, idc)

    rmin = jnp.min(val, axis=1, keepdims=True)
    lane = lax.broadcasted_iota(jnp.int32, (_NB, _C), 1).astype(jnp.float32)
    gidx = idc * _C + lane
    cand = jnp.where(val == rmin, gidx, _BIG)
    gmin = jnp.min(cand, axis=1, keepdims=True)
    qidx_ref[...] = gmin.astype(jnp.int32)
    loss_ref[0, 0] = _COMMIT * (jnp.sum(rmin) * (1.0 / (_N * _D)))


def _argmin_call(flat, embed, x2, e2r):
    return pl.pallas_call(
        _argmin_body,
        in_specs=[
            pl.BlockSpec(memory_space=pltpu.VMEM),
            pl.BlockSpec(memory_space=pltpu.VMEM),
            pl.BlockSpec(memory_space=pltpu.VMEM),
            pl.BlockSpec(memory_space=pltpu.VMEM),
        ],
        out_specs=[
            pl.BlockSpec(memory_space=pltpu.VMEM),
            pl.BlockSpec(memory_space=pltpu.SMEM),
        ],
        out_shape=[
            jax.ShapeDtypeStruct((_N, 1), jnp.int32),
            jax.ShapeDtypeStruct((1, 1), jnp.float32),
        ],
    )(flat, embed, x2, e2r)


_NS = 16
_BPW = _N // _NS
_CPW = _K // _NS
_TPAD = 4104


def _sc_body(idx_hbm, emb_hbm, zeros_hbm, ones_hbm, table_hbm,
             zq_hbm, perp_hbm,
             idx_v, rows_v, ones_v, cnt_v, table_v, ent_v, acc_v,
             counts_sh, ent_sh, sem, sem_w, sem_a, sem_b):
    s = lax.axis_index("s")
    base = s * _BPW

    @pl.when(s == 0)
    def _():
        pltpu.sync_copy(zeros_hbm, counts_sh)

    pltpu.sync_copy(idx_hbm.at[pl.ds(base, _BPW)], idx_v)
    gat = pltpu.async_copy(emb_hbm.at[idx_v], rows_v, sem)
    h_ones = pltpu.async_copy(ones_hbm, ones_v, sem_a)
    h_tab = pltpu.async_copy(table_hbm, table_v, sem_b)
    gat.wait()
    zqw = pltpu.async_copy(rows_v, zq_hbm.at[pl.ds(base, _BPW)], sem_w)
    h_ones.wait()
    h_tab.wait()
    plsc.subcore_barrier()
    pltpu.sync_copy(ones_v, counts_sh.at[idx_v], add=True)
    plsc.subcore_barrier()
    pltpu.sync_copy(counts_sh.at[pl.ds(s * _CPW, _CPW)], cnt_v)
    acc = jnp.zeros((16,), jnp.float32)
    for j in range(_CPW // 16):
        ci = cnt_v[pl.ds(j * 16, 16)].astype(jnp.int32)
        acc = acc + plsc.load_gather(table_v, [ci])
    ent_v[...] = acc
    pltpu.sync_copy(ent_v, ent_sh.at[s])
    plsc.subcore_barrier()

    @pl.when(s == 0)
    def _():
        pltpu.sync_copy(ent_sh, acc_v)
        tot = jnp.zeros((16,), jnp.float32)
        for r in range(_NS):
            tot = tot + acc_v[r]
        ent = jnp.sum(tot)
        ent_v[...] = jnp.exp(jnp.broadcast_to(ent, (16,)))
        pltpu.sync_copy(ent_v, perp_hbm)

    zqw.wait()


@functools.cache
def _sc_gather_hist():
    return pl.kernel(
        _sc_body,
        mesh=plsc.VectorSubcoreMesh(
            core_axis_name="c", subcore_axis_name="s", num_cores=1),
        compiler_params=pltpu.CompilerParams(
            use_tc_tiling_on_sc=False, needs_layout_passes=False),
        out_type=[
            jax.ShapeDtypeStruct((_N, _D), jnp.float32),
            jax.ShapeDtypeStruct((16,), jnp.float32),
        ],
        scratch_types=[
            pltpu.VMEM((_BPW,), jnp.int32),
            pltpu.VMEM((_BPW, _D), jnp.float32),
            pltpu.VMEM((_BPW,), jnp.float32),
            pltpu.VMEM((_CPW,), jnp.float32),
            pltpu.VMEM((_TPAD,), jnp.float32),
            pltpu.VMEM((16,), jnp.float32),
            pltpu.VMEM((_NS, 16), jnp.float32),
            pltpu.VMEM_SHARED((_K,), jnp.float32),
            pltpu.VMEM_SHARED((_NS, 16), jnp.float32),
            pltpu.SemaphoreType.DMA,
            pltpu.SemaphoreType.DMA,
            pltpu.SemaphoreType.DMA,
            pltpu.SemaphoreType.DMA,
        ],
    )


_KLDIV_VAL = np.log(float(_K)) * (_N / 4)


def kernel(inputs, embed):
    x = jnp.swapaxes(inputs, 1, -1)
    input_shape = x.shape
    flat = x.reshape(_N, _D)
    x2 = jnp.sum(flat * flat, axis=1, keepdims=True)
    e2 = jnp.sum(embed * embed, axis=1)
    e2r = e2.reshape(_K // _KB, 1, _KB)

    qidx, loss = _argmin_call(flat, embed, x2, e2r)

    zeros = jnp.zeros((_K,), jnp.float32)
    ones = jnp.ones((_BPW,), jnp.float32)
    cgrid = jnp.arange(_TPAD, dtype=jnp.float32) * (1.0 / _N)
    table = -(cgrid * jnp.log(cgrid + 1e-10)).at[0].set(0.0)
    zq, perp_v = _sc_gather_hist()(qidx.reshape(_N), embed, zeros, ones, table)
    perp = perp_v[0].reshape(1, 1)

    zq_r = zq.astype(jnp.bfloat16).astype(jnp.float32)
    z_q_out = jnp.swapaxes(zq_r.reshape(input_shape), 1, -1)
    kldiv_r = jnp.full((inputs.shape[0], 1), _KLDIV_VAL, jnp.float32)
    return (z_q_out, loss[0, 0], kldiv_r, perp[0, 0])
